# baseline (speedup 1.0000x reference)
import functools

import jax
import jax.numpy as jnp
from jax import lax
from jax.experimental import pallas as pl
from jax.experimental.pallas import tpu as pltpu
from jax.experimental.pallas import tpu_sc as plsc

R = 1024
V = 100000
B = 16384
NC, NS = 2, 16
NW = NC * NS
NCHUNK = 8
CB = B // NCHUNK
BW = CB // NW
CK = 16
NCK = BW // CK


def _sc_body(tt_hbm, idx_hbm, out_hbm, idx_v, rb0, rb1, gs0, gs1, ws0, ws1):
    cid = lax.axis_index("c")
    sid = lax.axis_index("s")
    wid = sid * NC + cid
    base = wid * BW
    rb = (rb0, rb1)
    gsem = (gs0, gs1)
    wsem = (ws0, ws1)

    pltpu.sync_copy(idx_hbm.at[pl.ds(base, BW)], idx_v)

    def gather_start(c):
        pltpu.make_async_copy(
            tt_hbm.at[idx_v.at[pl.ds(c * CK, CK)]], rb[c % 2],
            gsem[c % 2]).start()

    def write_start(c):
        pltpu.make_async_copy(
            rb[c % 2], out_hbm.at[pl.ds(base + c * CK, CK)],
            wsem[c % 2]).start()

    gather_start(0)
    for c in range(NCK):
        par = c % 2
        pltpu.make_async_copy(
            tt_hbm.at[pl.ds(0, CK)], rb[par], gsem[par]).wait()
        if c + 1 < NCK:
            if c >= 1:
                pltpu.make_async_copy(
                    tt_hbm.at[pl.ds(0, CK)], rb[1 - par], wsem[1 - par]).wait()
            gather_start(c + 1)
        write_start(c)
    pltpu.make_async_copy(tt_hbm.at[pl.ds(0, CK)], rb0, wsem[0]).wait()
    pltpu.make_async_copy(tt_hbm.at[pl.ds(0, CK)], rb1, wsem[1]).wait()


@functools.partial(
    pl.kernel,
    mesh=plsc.VectorSubcoreMesh(core_axis_name="c", subcore_axis_name="s"),
    out_type=jax.ShapeDtypeStruct((CB, R), jnp.float32),
    scratch_types=[
        pltpu.VMEM((BW,), jnp.int32),
        pltpu.VMEM((CK, R), jnp.float32),
        pltpu.VMEM((CK, R), jnp.float32),
        pltpu.SemaphoreType.DMA,
        pltpu.SemaphoreType.DMA,
        pltpu.SemaphoreType.DMA,
        pltpu.SemaphoreType.DMA,
    ],
    compiler_params=pltpu.CompilerParams(needs_layout_passes=False),
)
def _gather_rows(*refs):
    _sc_body(*refs)


TBLK = 512


def _t_first_body(x_ref, o_ref):
    o_ref[...] = x_ref[...].T


def _t_update_body(acc_ref, x_ref, o_ref):
    del acc_ref
    o_ref[...] = x_ref[...].T


def _tc_transpose_first(blk):
    return pl.pallas_call(
        _t_first_body,
        grid=(CB // TBLK,),
        in_specs=[pl.BlockSpec((TBLK, R), lambda i: (i, 0))],
        out_specs=pl.BlockSpec((R, TBLK), lambda i: (0, i)),
        out_shape=jax.ShapeDtypeStruct((R, B), jnp.float32),
    )(blk)


def _tc_transpose_update(acc, blk, k):
    return pl.pallas_call(
        _t_update_body,
        grid=(CB // TBLK,),
        in_specs=[
            pl.BlockSpec(memory_space=pl.MemorySpace.ANY),
            pl.BlockSpec((TBLK, R), lambda i: (i, 0)),
        ],
        out_specs=pl.BlockSpec(
            (R, TBLK), lambda i, k=k: (0, k * (CB // TBLK) + i)),
        out_shape=jax.ShapeDtypeStruct((R, B), jnp.float32),
        input_output_aliases={0: 0},
    )(acc, blk)


def kernel(t, index):
    tt = t.T
    idx = index.astype(jnp.int32)
    blks = [
        _gather_rows(tt, lax.slice(idx, (k * CB,), ((k + 1) * CB,)))
        for k in range(NCHUNK)
    ]
    out = _tc_transpose_first(blks[0])
    for k in range(1, NCHUNK):
        out = _tc_transpose_update(out, blks[k], k)
    return out


# speedup vs baseline: 1.1489x; 1.1489x over previous
import functools

import jax
import jax.numpy as jnp
from jax import lax
from jax.experimental import pallas as pl
from jax.experimental.pallas import tpu as pltpu
from jax.experimental.pallas import tpu_sc as plsc

R = 1024
V = 100000
B = 16384
NC, NS = 2, 16
NW = NC * NS
NCHUNK = 4
CB = B // NCHUNK
BW = CB // NW
CK = 32
NCK = BW // CK


def _sc_body(tt_hbm, idx_hbm, out_hbm, idx_v, rb0, rb1, gs0, gs1, ws0, ws1):
    cid = lax.axis_index("c")
    sid = lax.axis_index("s")
    wid = sid * NC + cid
    base = wid * BW
    rb = (rb0, rb1)
    gsem = (gs0, gs1)
    wsem = (ws0, ws1)

    pltpu.sync_copy(idx_hbm.at[pl.ds(base, BW)], idx_v)

    def gather_start(c):
        pltpu.make_async_copy(
            tt_hbm.at[idx_v.at[pl.ds(c * CK, CK)]], rb[c % 2],
            gsem[c % 2]).start()

    def write_start(c):
        pltpu.make_async_copy(
            rb[c % 2], out_hbm.at[pl.ds(base + c * CK, CK)],
            wsem[c % 2]).start()

    gather_start(0)
    for c in range(NCK):
        par = c % 2
        pltpu.make_async_copy(
            tt_hbm.at[pl.ds(0, CK)], rb[par], gsem[par]).wait()
        if c + 1 < NCK:
            if c >= 1:
                pltpu.make_async_copy(
                    tt_hbm.at[pl.ds(0, CK)], rb[1 - par], wsem[1 - par]).wait()
            gather_start(c + 1)
        write_start(c)
    pltpu.make_async_copy(tt_hbm.at[pl.ds(0, CK)], rb0, wsem[0]).wait()
    pltpu.make_async_copy(tt_hbm.at[pl.ds(0, CK)], rb1, wsem[1]).wait()


@functools.partial(
    pl.kernel,
    mesh=plsc.VectorSubcoreMesh(core_axis_name="c", subcore_axis_name="s"),
    out_type=jax.ShapeDtypeStruct((CB, R), jnp.float32),
    scratch_types=[
        pltpu.VMEM((BW,), jnp.int32),
        pltpu.VMEM((CK, R), jnp.float32),
        pltpu.VMEM((CK, R), jnp.float32),
        pltpu.SemaphoreType.DMA,
        pltpu.SemaphoreType.DMA,
        pltpu.SemaphoreType.DMA,
        pltpu.SemaphoreType.DMA,
    ],
    compiler_params=pltpu.CompilerParams(needs_layout_passes=False),
)
def _gather_rows(*refs):
    _sc_body(*refs)


TBLK = 1024


def _t_first_body(x_ref, o_ref):
    o_ref[...] = x_ref[...].T


def _t_update_body(acc_ref, x_ref, o_ref):
    del acc_ref
    o_ref[...] = x_ref[...].T


def _tc_transpose_first(blk):
    return pl.pallas_call(
        _t_first_body,
        grid=(CB // TBLK,),
        in_specs=[pl.BlockSpec((TBLK, R), lambda i: (i, 0))],
        out_specs=pl.BlockSpec((R, TBLK), lambda i: (0, i)),
        out_shape=jax.ShapeDtypeStruct((R, B), jnp.float32),
    )(blk)


def _tc_transpose_update(acc, blk, k):
    return pl.pallas_call(
        _t_update_body,
        grid=(CB // TBLK,),
        in_specs=[
            pl.BlockSpec(memory_space=pl.MemorySpace.ANY),
            pl.BlockSpec((TBLK, R), lambda i: (i, 0)),
        ],
        out_specs=pl.BlockSpec(
            (R, TBLK), lambda i, k=k: (0, k * (CB // TBLK) + i)),
        out_shape=jax.ShapeDtypeStruct((R, B), jnp.float32),
        input_output_aliases={0: 0},
    )(acc, blk)


def kernel(t, index):
    tt = t.T
    idx = index.astype(jnp.int32)
    blks = [
        _gather_rows(tt, lax.slice(idx, (k * CB,), ((k + 1) * CB,)))
        for k in range(NCHUNK)
    ]
    out = _tc_transpose_first(blks[0])
    for k in range(1, NCHUNK):
        out = _tc_transpose_update(out, blks[k], k)
    return out


# speedup vs baseline: 1.1538x; 1.0043x over previous
import functools

import jax
import jax.numpy as jnp
from jax import lax
from jax.experimental import pallas as pl
from jax.experimental.pallas import tpu as pltpu
from jax.experimental.pallas import tpu_sc as plsc

R = 1024
V = 100000
B = 16384
NC, NS = 2, 16
NW = NC * NS
CK = 32
CHUNKS = (2048, 6144, 6144, 2048)
TBLK = 1024


def _make_sc_body(bw, nck):
    def _sc_body(tt_hbm, idx_hbm, out_hbm, idx_v, rb0, rb1,
                 gs0, gs1, ws0, ws1):
        cid = lax.axis_index("c")
        sid = lax.axis_index("s")
        wid = sid * NC + cid
        base = wid * bw
        rb = (rb0, rb1)
        gsem = (gs0, gs1)
        wsem = (ws0, ws1)

        pltpu.sync_copy(idx_hbm.at[pl.ds(base, bw)], idx_v)

        def gather_start(c):
            pltpu.make_async_copy(
                tt_hbm.at[idx_v.at[pl.ds(c * CK, CK)]], rb[c % 2],
                gsem[c % 2]).start()

        def write_start(c):
            pltpu.make_async_copy(
                rb[c % 2], out_hbm.at[pl.ds(base + c * CK, CK)],
                wsem[c % 2]).start()

        gather_start(0)
        for c in range(nck):
            par = c % 2
            pltpu.make_async_copy(
                tt_hbm.at[pl.ds(0, CK)], rb[par], gsem[par]).wait()
            if c + 1 < nck:
                if c >= 1:
                    pltpu.make_async_copy(
                        tt_hbm.at[pl.ds(0, CK)], rb[1 - par],
                        wsem[1 - par]).wait()
                gather_start(c + 1)
            write_start(c)
        pltpu.make_async_copy(tt_hbm.at[pl.ds(0, CK)], rb0, wsem[0]).wait()
        pltpu.make_async_copy(tt_hbm.at[pl.ds(0, CK)], rb1, wsem[1]).wait()

    return _sc_body


@functools.cache
def _make_gather(cb):
    bw = cb // NW
    nck = bw // CK
    body = _make_sc_body(bw, nck)
    return pl.kernel(
        lambda *refs: body(*refs),
        mesh=plsc.VectorSubcoreMesh(core_axis_name="c", subcore_axis_name="s"),
        out_type=jax.ShapeDtypeStruct((cb, R), jnp.float32),
        scratch_types=[
            pltpu.VMEM((bw,), jnp.int32),
            pltpu.VMEM((CK, R), jnp.float32),
            pltpu.VMEM((CK, R), jnp.float32),
            pltpu.SemaphoreType.DMA,
            pltpu.SemaphoreType.DMA,
            pltpu.SemaphoreType.DMA,
            pltpu.SemaphoreType.DMA,
        ],
        compiler_params=pltpu.CompilerParams(needs_layout_passes=False),
    )


def _t_first_body(x_ref, o_ref):
    o_ref[...] = x_ref[...].T


def _t_update_body(acc_ref, x_ref, o_ref):
    del acc_ref
    o_ref[...] = x_ref[...].T


def _tc_transpose_first(blk):
    cb = blk.shape[0]
    return pl.pallas_call(
        _t_first_body,
        grid=(cb // TBLK,),
        in_specs=[pl.BlockSpec((TBLK, R), lambda i: (i, 0))],
        out_specs=pl.BlockSpec((R, TBLK), lambda i: (0, i)),
        out_shape=jax.ShapeDtypeStruct((R, B), jnp.float32),
    )(blk)


def _tc_transpose_update(acc, blk, col0):
    cb = blk.shape[0]
    blk0 = col0 // TBLK
    return pl.pallas_call(
        _t_update_body,
        grid=(cb // TBLK,),
        in_specs=[
            pl.BlockSpec(memory_space=pl.MemorySpace.ANY),
            pl.BlockSpec((TBLK, R), lambda i: (i, 0)),
        ],
        out_specs=pl.BlockSpec(
            (R, TBLK), lambda i, blk0=blk0: (0, blk0 + i)),
        out_shape=jax.ShapeDtypeStruct((R, B), jnp.float32),
        input_output_aliases={0: 0},
    )(acc, blk)


def kernel(t, index):
    tt = t.T
    idx = index.astype(jnp.int32)
    blks = []
    off = 0
    for cb in CHUNKS:
        blks.append(_make_gather(cb)(tt, lax.slice(idx, (off,), (off + cb,))))
        off += cb
    out = _tc_transpose_first(blks[0])
    col0 = CHUNKS[0]
    for k in range(1, len(CHUNKS)):
        out = _tc_transpose_update(out, blks[k], col0)
        col0 += CHUNKS[k]
    return out
